# baseline (speedup 1.0000x reference)
import functools

import jax
import jax.numpy as jnp
from jax import lax
from jax.experimental import pallas as pl
from jax.experimental.pallas import tpu as pltpu
from jax.experimental.pallas import tpu_sc as plsc

_BLANK = 0
_T = 8192
_L = 29
_PAD = 16
_NSUB = 16
_CHUNK = _T // _NSUB
_WIN = _CHUNK + _PAD
_NVEC = _CHUNK // 16

_MESH = plsc.VectorSubcoreMesh(core_axis_name="c", subcore_axis_name="s")
_CPARAMS = pltpu.CompilerParams(needs_layout_passes=False)


def _stage1_body(eflat, chunks_out, counts_out, buf, idxb, locout, cvec):
    cid = lax.axis_index("c")

    @pl.when(cid == 0)
    def _():
        wid = lax.axis_index("s")
        iota = lax.broadcasted_iota(jnp.int32, (16,), 0)
        riota = iota * _L

        first = wid == 0
        src = jnp.where(first, 0, wid * (_CHUNK * _L) - _PAD * _L)
        start = jnp.where(first, 0, _PAD)
        pltpu.sync_copy(eflat.at[pl.ds(src, _WIN * _L)], buf)

        def argmax_vec(j, _):
            colbase = j * (16 * _L)
            best = plsc.load_gather(buf, [riota + colbase])
            bidx = jnp.zeros((16,), jnp.int32)
            for r in range(1, _L):
                v = plsc.load_gather(buf, [riota + (colbase + r)])
                gt = v > best
                best = jnp.where(gt, v, best)
                bidx = jnp.where(gt, r, bidx)
            idxb[pl.ds(j * 16, 16)] = bidx
            return 0

        lax.fori_loop(0, _WIN // 16, argmax_vec, 0)

        def init_vec(j, _):
            locout[pl.ds(j * 16, 16)] = jnp.full((16,), -1, jnp.int32)
            return 0

        lax.fori_loop(0, _NVEC, init_vec, 0)

        def dedup_vec(j, cnt):
            base = start + j * 16
            cur = plsc.load_gather(idxb, [iota + base])
            prev = plsc.load_gather(idxb, [jnp.maximum(iota + base - 1, 0)])
            bmask = first & (j == 0) & (iota == 0)
            prev = jnp.where(bmask, -1, prev)
            keep = (cur != prev) & (cur != _BLANK)
            ki = keep.astype(jnp.int32)
            pos = cnt + plsc.cumsum(ki) - 1
            plsc.store_scatter(locout, [pos], cur, mask=keep)
            return cnt + jnp.sum(ki)

        cnt = lax.fori_loop(0, _NVEC, dedup_vec, jnp.int32(0))

        pltpu.sync_copy(locout, chunks_out.at[pl.ds(wid * _CHUNK, _CHUNK)])
        cvec[...] = jnp.zeros((16,), jnp.int32) + cnt
        pltpu.sync_copy(cvec, counts_out.at[pl.ds(wid * 16, 16)])


_stage1 = functools.partial(
    pl.kernel,
    out_type=[
        jax.ShapeDtypeStruct((_T,), jnp.int32),
        jax.ShapeDtypeStruct((_NSUB * 16,), jnp.int32),
    ],
    mesh=_MESH,
    compiler_params=_CPARAMS,
    scratch_types=[
        pltpu.VMEM((_WIN * _L,), jnp.float32),
        pltpu.VMEM((_WIN,), jnp.int32),
        pltpu.VMEM((_CHUNK,), jnp.int32),
        pltpu.VMEM((16,), jnp.int32),
    ],
)(_stage1_body)


def _stage2_body(chunks, counts_in, packed_out, cnt_out, allc, cntv, outv,
                 offs, cvec):
    cid = lax.axis_index("c")

    @pl.when(cid == 0)
    def _():
        wid = lax.axis_index("s")
        iota = lax.broadcasted_iota(jnp.int32, (16,), 0)

        pltpu.sync_copy(chunks.at[pl.ds(0, _T)], allc)
        pltpu.sync_copy(counts_in.at[pl.ds(0, _NSUB * 16)], cntv)
        counts = plsc.load_gather(cntv, [iota * 16])
        total = jnp.sum(counts)
        offs[...] = plsc.cumsum(counts) - counts

        def pack_vec(j, _):
            p = wid * _CHUNK + j * 16 + iota
            acc = jnp.zeros((16,), jnp.int32)

            def count_le(k, a):
                offk = plsc.load_gather(
                    offs, [jnp.zeros((16,), jnp.int32) + k])
                return a + (p >= offk).astype(jnp.int32)

            acc = lax.fori_loop(0, _NSUB, count_le, acc)
            w = acc - 1
            myoff = plsc.load_gather(offs, [w])
            local = jnp.minimum(p - myoff, _CHUNK - 1)
            val = plsc.load_gather(allc, [w * _CHUNK + local])
            outv[pl.ds(j * 16, 16)] = jnp.where(p < total, val, -1)
            return 0

        lax.fori_loop(0, _NVEC, pack_vec, 0)
        pltpu.sync_copy(outv, packed_out.at[pl.ds(wid * _CHUNK, _CHUNK)])

        @pl.when(wid == 0)
        def _():
            cvec[...] = jnp.zeros((16,), jnp.int32) + total
            pltpu.sync_copy(cvec, cnt_out)


_stage2 = functools.partial(
    pl.kernel,
    out_type=[
        jax.ShapeDtypeStruct((_T,), jnp.int32),
        jax.ShapeDtypeStruct((16,), jnp.int32),
    ],
    mesh=_MESH,
    compiler_params=_CPARAMS,
    scratch_types=[
        pltpu.VMEM((_T,), jnp.int32),
        pltpu.VMEM((_NSUB * 16,), jnp.int32),
        pltpu.VMEM((_CHUNK,), jnp.int32),
        pltpu.VMEM((16,), jnp.int32),
        pltpu.VMEM((16,), jnp.int32),
    ],
)(_stage2_body)


def kernel(emissions):
    eflat = emissions.reshape(-1)
    chunks, counts = _stage1(eflat)
    packed, cnt16 = _stage2(chunks, counts)
    return packed, cnt16[0]


# speedup vs baseline: 4.3668x; 4.3668x over previous
import functools

import jax
import jax.numpy as jnp
from jax import lax
from jax.experimental import pallas as pl
from jax.experimental.pallas import tpu as pltpu
from jax.experimental.pallas import tpu_sc as plsc

_BLANK = 0
_T = 8192
_L = 29
_PAD = 16
_NSUB = 16
_CHUNK = _T // _NSUB
_WIN = _CHUNK + _PAD
_NVEC = _CHUNK // 16

_MESH = plsc.VectorSubcoreMesh(core_axis_name="c", subcore_axis_name="s")
_CPARAMS = pltpu.CompilerParams(needs_layout_passes=False)


def _stage1_body(eflat, chunks_out, counts_out, buf, idxb, locout, cvec):
    cid = lax.axis_index("c")

    @pl.when(cid == 0)
    def _():
        wid = lax.axis_index("s")
        iota = lax.broadcasted_iota(jnp.int32, (16,), 0)
        riota = iota * _L

        first = wid == 0
        src = jnp.where(first, 0, wid * (_CHUNK * _L) - _PAD * _L)
        start = jnp.where(first, 0, _PAD)
        pltpu.sync_copy(eflat.at[pl.ds(src, _WIN * _L)], buf)

        def argmax_vec(j, _):
            colbase = j * (16 * _L)
            best = plsc.load_gather(buf, [riota + colbase])
            bidx = jnp.zeros((16,), jnp.int32)
            for r in range(1, _L):
                v = plsc.load_gather(buf, [riota + (colbase + r)])
                gt = v > best
                best = jnp.where(gt, v, best)
                bidx = jnp.where(gt, r, bidx)
            idxb[pl.ds(j * 16, 16)] = bidx
            return 0

        lax.fori_loop(0, _WIN // 16, argmax_vec, 0)

        def init_vec(j, _):
            locout[pl.ds(j * 16, 16)] = jnp.full((16,), -1, jnp.int32)
            return 0

        lax.fori_loop(0, _NVEC, init_vec, 0)

        def dedup_vec(j, cnt):
            base = start + j * 16
            cur = plsc.load_gather(idxb, [iota + base])
            prev = plsc.load_gather(idxb, [jnp.maximum(iota + base - 1, 0)])
            bmask = first & (j == 0) & (iota == 0)
            prev = jnp.where(bmask, -1, prev)
            keep = (cur != prev) & (cur != _BLANK)
            ki = keep.astype(jnp.int32)
            pos = cnt + plsc.cumsum(ki) - 1
            plsc.store_scatter(locout, [pos], cur, mask=keep)
            return cnt + jnp.sum(ki)

        cnt = lax.fori_loop(0, _NVEC, dedup_vec, jnp.int32(0))

        pltpu.sync_copy(locout, chunks_out.at[pl.ds(wid * _CHUNK, _CHUNK)])
        cvec[...] = jnp.zeros((16,), jnp.int32) + cnt
        pltpu.sync_copy(cvec, counts_out.at[pl.ds(wid * 16, 16)])


_stage1 = functools.partial(
    pl.kernel,
    out_type=[
        jax.ShapeDtypeStruct((_T,), jnp.int32),
        jax.ShapeDtypeStruct((_NSUB * 16,), jnp.int32),
    ],
    mesh=_MESH,
    compiler_params=_CPARAMS,
    scratch_types=[
        pltpu.VMEM((_WIN * _L,), jnp.float32),
        pltpu.VMEM((_WIN,), jnp.int32),
        pltpu.VMEM((_CHUNK,), jnp.int32),
        pltpu.VMEM((16,), jnp.int32),
    ],
)(_stage1_body)


def _stage2_body(chunks, counts_in, packed_out, cnt_out, allc, cntv, outv,
                 offs, cvec):
    cid = lax.axis_index("c")

    @pl.when(cid == 0)
    def _():
        wid = lax.axis_index("s")
        iota = lax.broadcasted_iota(jnp.int32, (16,), 0)

        pltpu.sync_copy(chunks.at[pl.ds(0, _T)], allc)
        pltpu.sync_copy(counts_in.at[pl.ds(0, _NSUB * 16)], cntv)
        counts = plsc.load_gather(cntv, [iota * 16])
        total = jnp.sum(counts)
        offs[...] = plsc.cumsum(counts) - counts

        def pack_vec(j, _):
            p = wid * _CHUNK + j * 16 + iota
            acc = jnp.zeros((16,), jnp.int32)

            def count_le(k, a):
                offk = plsc.load_gather(
                    offs, [jnp.zeros((16,), jnp.int32) + k])
                return a + (p >= offk).astype(jnp.int32)

            acc = lax.fori_loop(0, _NSUB, count_le, acc)
            w = acc - 1
            myoff = plsc.load_gather(offs, [w])
            local = jnp.minimum(p - myoff, _CHUNK - 1)
            val = plsc.load_gather(allc, [w * _CHUNK + local])
            outv[pl.ds(j * 16, 16)] = jnp.where(p < total, val, -1)
            return 0

        lax.fori_loop(0, _NVEC, pack_vec, 0)
        pltpu.sync_copy(outv, packed_out.at[pl.ds(wid * _CHUNK, _CHUNK)])

        @pl.when(wid == 0)
        def _():
            cvec[...] = jnp.zeros((16,), jnp.int32) + total
            pltpu.sync_copy(cvec, cnt_out)


_stage2 = functools.partial(
    pl.kernel,
    out_type=[
        jax.ShapeDtypeStruct((_T,), jnp.int32),
        jax.ShapeDtypeStruct((16,), jnp.int32),
    ],
    mesh=_MESH,
    compiler_params=_CPARAMS,
    scratch_types=[
        pltpu.VMEM((_T,), jnp.int32),
        pltpu.VMEM((_NSUB * 16,), jnp.int32),
        pltpu.VMEM((_CHUNK,), jnp.int32),
        pltpu.VMEM((16,), jnp.int32),
        pltpu.VMEM((16,), jnp.int32),
    ],
)(_stage2_body)


def kernel(emissions):
    eflat = emissions[0].reshape(-1)
    chunks, counts = _stage1(eflat)
    packed, cnt16 = _stage2(chunks, counts)
    return packed, cnt16[0]


# speedup vs baseline: 5.1323x; 1.1753x over previous
import functools

import jax
import jax.numpy as jnp
from jax import lax
from jax.experimental import pallas as pl
from jax.experimental.pallas import tpu as pltpu
from jax.experimental.pallas import tpu_sc as plsc

_BLANK = 0
_T = 8192
_L = 29
_PAD = 16
_NSUB = 16
_CHUNK = _T // _NSUB
_WIN = _CHUNK + _PAD
_NVEC = _CHUNK // 16
_ROW = _CHUNK + 16


def _decode_body(eflat, packed_out, cnt_out, buf, idxb, locout, outv, cvec,
                 offs, allc, sh):
    cid = lax.axis_index("c")
    wid = lax.axis_index("s")
    iota = lax.broadcasted_iota(jnp.int32, (16,), 0)

    @pl.when(cid == 0)
    def _phase1():
        riota = iota * _L

        first = wid == 0
        src = jnp.where(first, 0, wid * (_CHUNK * _L) - _PAD * _L)
        start = jnp.where(first, 0, _PAD)
        pltpu.sync_copy(eflat.at[pl.ds(src, _WIN * _L)], buf)

        def argmax_vec(j, _):
            colbase = j * (16 * _L)
            best = plsc.load_gather(buf, [riota + colbase])
            bidx = jnp.zeros((16,), jnp.int32)
            for r in range(1, _L):
                v = plsc.load_gather(buf, [riota + (colbase + r)])
                gt = v > best
                best = jnp.where(gt, v, best)
                bidx = jnp.where(gt, r, bidx)
            idxb[pl.ds(j * 16, 16)] = bidx
            return 0

        lax.fori_loop(0, _WIN // 16, argmax_vec, 0)

        def init_vec(j, _):
            locout[pl.ds(j * 16, 16)] = jnp.full((16,), -1, jnp.int32)
            return 0

        lax.fori_loop(0, _NVEC, init_vec, 0)

        def dedup_vec(j, cnt):
            base = start + j * 16
            cur = plsc.load_gather(idxb, [iota + base])
            prev = plsc.load_gather(idxb, [jnp.maximum(iota + base - 1, 0)])
            bmask = first & (j == 0) & (iota == 0)
            prev = jnp.where(bmask, -1, prev)
            keep = (cur != prev) & (cur != _BLANK)
            ki = keep.astype(jnp.int32)
            pos = cnt + plsc.cumsum(ki) - 1
            plsc.store_scatter(locout, [pos], cur, mask=keep)
            return cnt + jnp.sum(ki)

        cnt = lax.fori_loop(0, _NVEC, dedup_vec, jnp.int32(0))

        pltpu.sync_copy(locout, sh.at[pl.ds(wid * _ROW, _CHUNK)])
        cvec[...] = jnp.zeros((16,), jnp.int32) + cnt
        pltpu.sync_copy(cvec, sh.at[pl.ds(wid * _ROW + _CHUNK, 16)])

    plsc.subcore_barrier()

    @pl.when(cid == 0)
    def _phase2():
        pltpu.sync_copy(sh.at[pl.ds(0, _NSUB * _ROW)], allc)
        counts = plsc.load_gather(allc, [iota * _ROW + _CHUNK])
        total = jnp.sum(counts)
        offs[...] = plsc.cumsum(counts) - counts

        def pack_vec(j, _):
            p = wid * _CHUNK + j * 16 + iota
            acc = jnp.zeros((16,), jnp.int32)

            def count_le(k, a):
                offk = plsc.load_gather(
                    offs, [jnp.zeros((16,), jnp.int32) + k])
                return a + (p >= offk).astype(jnp.int32)

            acc = lax.fori_loop(0, _NSUB, count_le, acc)
            w = acc - 1
            myoff = plsc.load_gather(offs, [w])
            local = jnp.minimum(p - myoff, _CHUNK - 1)
            val = plsc.load_gather(allc, [w * _ROW + local])
            outv[pl.ds(j * 16, 16)] = jnp.where(p < total, val, -1)
            return 0

        lax.fori_loop(0, _NVEC, pack_vec, 0)
        pltpu.sync_copy(outv, packed_out.at[pl.ds(wid * _CHUNK, _CHUNK)])

        @pl.when(wid == 0)
        def _():
            cvec[...] = jnp.zeros((16,), jnp.int32) + total
            pltpu.sync_copy(cvec, cnt_out)


_decode = functools.partial(
    pl.kernel,
    out_type=[
        jax.ShapeDtypeStruct((_T,), jnp.int32),
        jax.ShapeDtypeStruct((16,), jnp.int32),
    ],
    mesh=plsc.VectorSubcoreMesh(core_axis_name="c", subcore_axis_name="s"),
    compiler_params=pltpu.CompilerParams(needs_layout_passes=False),
    scratch_types=[
        pltpu.VMEM((_WIN * _L,), jnp.float32),
        pltpu.VMEM((_WIN,), jnp.int32),
        pltpu.VMEM((_CHUNK,), jnp.int32),
        pltpu.VMEM((_CHUNK,), jnp.int32),
        pltpu.VMEM((16,), jnp.int32),
        pltpu.VMEM((16,), jnp.int32),
        pltpu.VMEM((_NSUB * _ROW,), jnp.int32),
        pltpu.VMEM_SHARED((_NSUB * _ROW,), jnp.int32),
    ],
)(_decode_body)


def kernel(emissions):
    eflat = emissions[0].reshape(-1)
    packed, cnt16 = _decode(eflat)
    return packed, cnt16[0]


# speedup vs baseline: 5.4289x; 1.0578x over previous
import functools

import jax
import jax.numpy as jnp
from jax import lax
from jax.experimental import pallas as pl
from jax.experimental.pallas import tpu as pltpu
from jax.experimental.pallas import tpu_sc as plsc

_BLANK = 0
_T = 8192
_L = 29
_PAD = 16
_NSUB = 16
_CHUNK = _T // _NSUB
_WIN = _CHUNK + _PAD
_NVEC = _CHUNK // 16
_ROW = _CHUNK + 16


def _decode_body(eflat, packed_out, cnt_out, buf, idxb, locout, outv, cvec,
                 offs, allc, sh):
    cid = lax.axis_index("c")
    wid = lax.axis_index("s")
    iota = lax.broadcasted_iota(jnp.int32, (16,), 0)

    @pl.when(cid == 0)
    def _phase1():
        riota = iota * _L

        first = wid == 0
        src = jnp.where(first, 0, wid * (_CHUNK * _L) - _PAD * _L)
        start = jnp.where(first, 0, _PAD)
        pltpu.sync_copy(eflat.at[pl.ds(src, _WIN * _L)], buf)

        def argmax_vec(j, _):
            colbase = j * (16 * _L)
            best = plsc.load_gather(buf, [riota + colbase])
            bidx = jnp.zeros((16,), jnp.int32)
            for r in range(1, _L):
                v = plsc.load_gather(buf, [riota + (colbase + r)])
                gt = v > best
                best = jnp.where(gt, v, best)
                bidx = jnp.where(gt, r, bidx)
            idxb[pl.ds(j * 16, 16)] = bidx
            return 0

        lax.fori_loop(0, _WIN // 16, argmax_vec, 0)

        def init_vec(j, _):
            locout[pl.ds(j * 16, 16)] = jnp.full((16,), -1, jnp.int32)
            return 0

        lax.fori_loop(0, _NVEC, init_vec, 0)

        def dedup_vec(j, cnt):
            base = start + j * 16
            cur = plsc.load_gather(idxb, [iota + base])
            prev = plsc.load_gather(idxb, [jnp.maximum(iota + base - 1, 0)])
            bmask = first & (j == 0) & (iota == 0)
            prev = jnp.where(bmask, -1, prev)
            keep = (cur != prev) & (cur != _BLANK)
            ki = keep.astype(jnp.int32)
            pos = cnt + plsc.cumsum(ki) - 1
            plsc.store_scatter(locout, [pos], cur, mask=keep)
            return cnt + jnp.sum(ki)

        cnt = lax.fori_loop(0, _NVEC, dedup_vec, jnp.int32(0))

        pltpu.sync_copy(locout, sh.at[pl.ds(wid * _ROW, _CHUNK)])
        cvec[...] = jnp.zeros((16,), jnp.int32) + cnt
        pltpu.sync_copy(cvec, sh.at[pl.ds(wid * _ROW + _CHUNK, 16)])

    plsc.subcore_barrier()

    @pl.when(cid == 0)
    def _phase2():
        pltpu.sync_copy(sh.at[pl.ds(0, _NSUB * _ROW)], allc)
        counts = plsc.load_gather(allc, [iota * _ROW + _CHUNK])
        total = jnp.sum(counts)
        offs[...] = plsc.cumsum(counts) - counts

        def pack_vec(j, _):
            p = wid * _CHUNK + j * 16 + iota
            acc = jnp.zeros((16,), jnp.int32)
            for k in range(_NSUB):
                offk = plsc.load_gather(offs, [jnp.full((16,), k, jnp.int32)])
                acc += (p >= offk).astype(jnp.int32)
            w = acc - 1
            myoff = plsc.load_gather(offs, [w])
            local = jnp.minimum(p - myoff, _CHUNK - 1)
            val = plsc.load_gather(allc, [w * _ROW + local])
            outv[pl.ds(j * 16, 16)] = jnp.where(p < total, val, -1)
            return 0

        lax.fori_loop(0, _NVEC, pack_vec, 0)
        pltpu.sync_copy(outv, packed_out.at[pl.ds(wid * _CHUNK, _CHUNK)])

        @pl.when(wid == 0)
        def _():
            cvec[...] = jnp.zeros((16,), jnp.int32) + total
            pltpu.sync_copy(cvec, cnt_out)


_decode = functools.partial(
    pl.kernel,
    out_type=[
        jax.ShapeDtypeStruct((_T,), jnp.int32),
        jax.ShapeDtypeStruct((16,), jnp.int32),
    ],
    mesh=plsc.VectorSubcoreMesh(core_axis_name="c", subcore_axis_name="s"),
    compiler_params=pltpu.CompilerParams(needs_layout_passes=False),
    scratch_types=[
        pltpu.VMEM((_WIN * _L,), jnp.float32),
        pltpu.VMEM((_WIN,), jnp.int32),
        pltpu.VMEM((_CHUNK,), jnp.int32),
        pltpu.VMEM((_CHUNK,), jnp.int32),
        pltpu.VMEM((16,), jnp.int32),
        pltpu.VMEM((16,), jnp.int32),
        pltpu.VMEM((_NSUB * _ROW,), jnp.int32),
        pltpu.VMEM_SHARED((_NSUB * _ROW,), jnp.int32),
    ],
)(_decode_body)


def kernel(emissions):
    eflat = emissions[0].reshape(-1)
    packed, cnt16 = _decode(eflat)
    return packed, cnt16[0]
